# baseline (speedup 1.0000x reference)
import functools

import jax
import jax.numpy as jnp
from jax import lax
from jax.experimental import pallas as pl
from jax.experimental.pallas import tpu as pltpu
from jax.experimental.pallas import tpu_sc as plsc

_NC = 2
_NS = 16
_NW = _NC * _NS
_L = 16

_C = 640
_G = 128
_K = _C // _G
_B = 16000
_CPB = _B // _C
_RB = 2000

_ACOS_COEF = (1.5707288, -0.2121144, 0.0742610, -0.0187293)
_PI = 3.14159265358979


def _acos(c):
    t = jnp.abs(c)
    u = (1.0 - t).astype(jnp.float32)
    i = lax.bitcast_convert_type(u, jnp.int32)
    i = jnp.int32(0x5F3759DF) - (i >> 1)
    y = lax.bitcast_convert_type(i, jnp.float32)
    y = y * (1.5 - 0.5 * u * y * y)
    s = u * y
    p = jnp.float32(_ACOS_COEF[3])
    for a in _ACOS_COEF[2::-1]:
        p = p * t + jnp.float32(a)
    r = s * p
    return jnp.where(c < 0, jnp.float32(_PI) - r, r)


def kernel(distances, vec, angle_src, angle_dst):
    A = angle_src.shape[0]
    E = distances.shape[0]
    per_w = A // _NW
    n_blocks = per_w // _B
    rows_per_sub = E // _NS
    n_build = rows_per_sub // _RB
    mesh = plsc.VectorSubcoreMesh(core_axis_name="c", subcore_axis_name="s")

    row_types = [
        pltpu.VMEM((_C, 8), jnp.float32),
        pltpu.VMEM((_C, 8), jnp.float32),
        pltpu.SemaphoreType.DMA,
        pltpu.SemaphoreType.DMA,
    ]

    @functools.partial(
        pl.kernel,
        out_type=(jax.ShapeDtypeStruct((A,), jnp.float32),
                  jax.ShapeDtypeStruct((_NC, E, 8), jnp.float32)),
        mesh=mesh,
        scratch_types=[
            pltpu.VMEM((_B,), jnp.int32),
            pltpu.VMEM((_B,), jnp.int32),
            pltpu.VMEM((_B,), jnp.float32),
            pltpu.VMEM((_RB * 3,), jnp.float32),
            pltpu.VMEM((_RB,), jnp.float32),
            pltpu.VMEM((_RB, 8), jnp.float32),
        ] + row_types + row_types,
        compiler_params=pltpu.CompilerParams(
            needs_layout_passes=False, use_tc_tiling_on_sc=False),
    )
    def angle_kernel(dist_h, vec_h, src_h, dst_h, out_h, tab2_h,
                     si, di, ob, bv, bd, bt,
                     r1_a, r2_a, sem1_a, sem2_a,
                     r1_b, r2_b, sem1_b, sem2_b):
        core = lax.axis_index("c")
        sub = lax.axis_index("s")
        wid = sub * _NC + core
        base = wid * per_w
        bufs = ((r1_a, r2_a, sem1_a, sem2_a),
                (r1_b, r2_b, sem1_b, sem2_b))

        def build_batch(b, carry):
            roff = sub * rows_per_sub + b * _RB
            pltpu.sync_copy(vec_h.at[pl.ds(roff * 3, _RB * 3)], bv)
            pltpu.sync_copy(dist_h.at[pl.ds(roff, _RB)], bd)

            @plsc.parallel_loop(0, _RB // _L, unroll=4)
            def pack(j):
                lane = lax.broadcasted_iota(jnp.int32, (_L,), 0)
                rp0 = (lane + 0) // 3
                cp0 = (lane + 0) % 3
                rp1 = (lane + 16) // 3
                cp1 = (lane + 16) % 3
                rp2 = (lane + 32) // 3
                cp2 = (lane + 32) % 3
                c3 = jnp.full((_L,), 3, jnp.int32)
                r16 = j * _L
                v0 = bv[pl.ds(j * 48, _L)]
                v1 = bv[pl.ds(j * 48 + 16, _L)]
                v2 = bv[pl.ds(j * 48 + 32, _L)]
                d = bd[pl.ds(r16, _L)]
                plsc.store_scatter(bt, [rp0 + r16, cp0], v0)
                plsc.store_scatter(bt, [rp1 + r16, cp1], v1)
                plsc.store_scatter(bt, [rp2 + r16, cp2], v2)
                plsc.store_scatter(bt, [lane + r16, c3], d)

            pltpu.sync_copy(bt, tab2_h.at[core, pl.ds(roff, _RB)])
            return carry

        lax.fori_loop(0, n_build, build_batch, 0)
        plsc.subcore_barrier()
        tab_h = tab2_h.at[core]

        def stage(ci, buf):
            r1, r2, sem1, sem2 = buf

            def fire(g, carry):
                isl = pl.ds(ci * _C + g * _G, _G)
                sl = pl.ds(g * _G, _G)
                pltpu.async_copy(tab_h.at[si.at[isl]], r1.at[sl], sem1)
                pltpu.async_copy(tab_h.at[di.at[isl]], r2.at[sl], sem2)
                return carry

            lax.fori_loop(0, _K, fire, 0)

        def finish(ci, buf):
            r1, r2, sem1, sem2 = buf

            def drain(g, carry):
                isl = pl.ds(ci * _C + g * _G, _G)
                sl = pl.ds(g * _G, _G)
                pltpu.make_async_copy(
                    tab_h.at[si.at[isl]], r1.at[sl], sem1).wait()
                pltpu.make_async_copy(
                    tab_h.at[di.at[isl]], r2.at[sl], sem2).wait()
                return carry

            lax.fori_loop(0, _K, drain, 0)

            @plsc.parallel_loop(0, _C // _L, unroll=8)
            def comp(j):
                rid = lax.broadcasted_iota(jnp.int32, (_L,), 0) + j * _L

                def ld(ref, c):
                    return plsc.load_gather(
                        ref, [rid, jnp.full((_L,), c, jnp.int32)])

                x1 = ld(r1, 0)
                y1 = ld(r1, 1)
                z1 = ld(r1, 2)
                d1 = ld(r1, 3)
                x2 = ld(r2, 0)
                y2 = ld(r2, 1)
                z2 = ld(r2, 2)
                d2 = ld(r2, 3)
                num = x1 * x2 + y1 * y2 + z1 * z2
                den = jnp.maximum(d1 * d2, jnp.float32(1e-10))
                ri = jnp.int32(0x7EF311C3) - lax.bitcast_convert_type(
                    den, jnp.int32)
                inv = lax.bitcast_convert_type(ri, jnp.float32)
                for _ in range(2):
                    inv = inv * (2.0 - den * inv)
                cosang = jnp.float32(0.95) * num * inv
                ob[pl.ds(ci * _C + j * _L, _L)] = _acos(cosang)

        def block_body(bi, carry):
            boff = base + bi * _B
            pltpu.sync_copy(src_h.at[pl.ds(boff, _B)], si)
            pltpu.sync_copy(dst_h.at[pl.ds(boff, _B)], di)

            stage(0, bufs[0])

            def ring(i, c2):
                ci0 = i * 2
                stage(ci0 + 1, bufs[1])
                finish(ci0, bufs[0])
                stage(ci0 + 2, bufs[0])
                finish(ci0 + 1, bufs[1])
                return c2

            lax.fori_loop(0, (_CPB - 1) // 2, ring, 0)
            finish(_CPB - 1, bufs[0])

            pltpu.sync_copy(ob, out_h.at[pl.ds(boff, _B)])
            return carry

        lax.fori_loop(0, n_blocks, block_body, 0)

    return angle_kernel(distances, vec.reshape(-1), angle_src, angle_dst)[0]


# speedup vs baseline: 1.1928x; 1.1928x over previous
import functools

import jax
import jax.numpy as jnp
from jax import lax
from jax.experimental import pallas as pl
from jax.experimental.pallas import tpu as pltpu
from jax.experimental.pallas import tpu_sc as plsc

_NC = 2
_NS = 16
_NW = _NC * _NS
_L = 16

_C = 640
_G = 128
_K = _C // _G
_B = 3200
_CPB = _B // _C
_RB = 1000

_ACOS_COEF = (1.5707288, -0.2121144, 0.0742610, -0.0187293)
_PI = 3.14159265358979


def _acos(c):
    t = jnp.abs(c)
    u = (1.0 - t).astype(jnp.float32)
    i = lax.bitcast_convert_type(u, jnp.int32)
    i = jnp.int32(0x5F3759DF) - (i >> 1)
    y = lax.bitcast_convert_type(i, jnp.float32)
    y = y * (1.5 - 0.5 * u * y * y)
    s = u * y
    p = jnp.float32(_ACOS_COEF[3])
    for a in _ACOS_COEF[2::-1]:
        p = p * t + jnp.float32(a)
    r = s * p
    return jnp.where(c < 0, jnp.float32(_PI) - r, r)


def kernel(distances, vec, angle_src, angle_dst):
    A = angle_src.shape[0]
    E = distances.shape[0]
    per_w = A // _NW
    n_blocks = per_w // _B
    rows_per_sub = E // _NS
    n_build = rows_per_sub // _RB
    mesh = plsc.VectorSubcoreMesh(core_axis_name="c", subcore_axis_name="s")

    row_types = [
        pltpu.VMEM((_C, 8), jnp.float32),
        pltpu.VMEM((_C, 8), jnp.float32),
        pltpu.SemaphoreType.DMA,
        pltpu.SemaphoreType.DMA,
    ]

    @functools.partial(
        pl.kernel,
        out_type=jax.ShapeDtypeStruct((A,), jnp.float32),
        mesh=mesh,
        scratch_types=[
            pltpu.VMEM_SHARED((E, 8), jnp.float32),
            pltpu.VMEM((_B,), jnp.int32),
            pltpu.VMEM((_B,), jnp.int32),
            pltpu.VMEM((_B,), jnp.float32),
            pltpu.VMEM((_RB * 3,), jnp.float32),
            pltpu.VMEM((_RB,), jnp.float32),
            pltpu.VMEM((_RB, 8), jnp.float32),
        ] + row_types + row_types,
        compiler_params=pltpu.CompilerParams(
            needs_layout_passes=False, use_tc_tiling_on_sc=False),
    )
    def angle_kernel(dist_h, vec_h, src_h, dst_h, out_h,
                     tab_h, si, di, ob, bv, bd, bt,
                     r1_a, r2_a, sem1_a, sem2_a,
                     r1_b, r2_b, sem1_b, sem2_b):
        core = lax.axis_index("c")
        sub = lax.axis_index("s")
        wid = sub * _NC + core
        base = wid * per_w
        bufs = ((r1_a, r2_a, sem1_a, sem2_a),
                (r1_b, r2_b, sem1_b, sem2_b))

        def build_batch(b, carry):
            roff = sub * rows_per_sub + b * _RB
            pltpu.sync_copy(vec_h.at[pl.ds(roff * 3, _RB * 3)], bv)
            pltpu.sync_copy(dist_h.at[pl.ds(roff, _RB)], bd)

            @plsc.parallel_loop(0, _RB // _L, unroll=4)
            def pack(j):
                lane = lax.broadcasted_iota(jnp.int32, (_L,), 0)
                rp0 = (lane + 0) // 3
                cp0 = (lane + 0) % 3
                rp1 = (lane + 16) // 3
                cp1 = (lane + 16) % 3
                rp2 = (lane + 32) // 3
                cp2 = (lane + 32) % 3
                c3 = jnp.full((_L,), 3, jnp.int32)
                r16 = j * _L
                v0 = bv[pl.ds(j * 48, _L)]
                v1 = bv[pl.ds(j * 48 + 16, _L)]
                v2 = bv[pl.ds(j * 48 + 32, _L)]
                d = bd[pl.ds(r16, _L)]
                plsc.store_scatter(bt, [rp0 + r16, cp0], v0)
                plsc.store_scatter(bt, [rp1 + r16, cp1], v1)
                plsc.store_scatter(bt, [rp2 + r16, cp2], v2)
                plsc.store_scatter(bt, [lane + r16, c3], d)

            pltpu.sync_copy(bt, tab_h.at[pl.ds(roff, _RB)])
            return carry

        lax.fori_loop(0, n_build, build_batch, 0)
        plsc.subcore_barrier()

        def stage(ci, buf):
            r1, r2, sem1, sem2 = buf

            def fire(g, carry):
                isl = pl.ds(ci * _C + g * _G, _G)
                sl = pl.ds(g * _G, _G)
                pltpu.async_copy(tab_h.at[si.at[isl]], r1.at[sl], sem1)
                pltpu.async_copy(tab_h.at[di.at[isl]], r2.at[sl], sem2)
                return carry

            lax.fori_loop(0, _K, fire, 0)

        def finish(ci, buf):
            r1, r2, sem1, sem2 = buf

            def drain(g, carry):
                isl = pl.ds(ci * _C + g * _G, _G)
                sl = pl.ds(g * _G, _G)
                pltpu.make_async_copy(
                    tab_h.at[si.at[isl]], r1.at[sl], sem1).wait()
                pltpu.make_async_copy(
                    tab_h.at[di.at[isl]], r2.at[sl], sem2).wait()
                return carry

            lax.fori_loop(0, _K, drain, 0)

            @plsc.parallel_loop(0, _C // _L, unroll=8)
            def comp(j):
                rid = lax.broadcasted_iota(jnp.int32, (_L,), 0) + j * _L

                def ld(ref, c):
                    return plsc.load_gather(
                        ref, [rid, jnp.full((_L,), c, jnp.int32)])

                x1 = ld(r1, 0)
                y1 = ld(r1, 1)
                z1 = ld(r1, 2)
                d1 = ld(r1, 3)
                x2 = ld(r2, 0)
                y2 = ld(r2, 1)
                z2 = ld(r2, 2)
                d2 = ld(r2, 3)
                num = x1 * x2 + y1 * y2 + z1 * z2
                den = jnp.maximum(d1 * d2, jnp.float32(1e-10))
                ri = jnp.int32(0x7EF311C3) - lax.bitcast_convert_type(
                    den, jnp.int32)
                inv = lax.bitcast_convert_type(ri, jnp.float32)
                for _ in range(2):
                    inv = inv * (2.0 - den * inv)
                cosang = jnp.float32(0.95) * num * inv
                ob[pl.ds(ci * _C + j * _L, _L)] = _acos(cosang)

        def block_body(bi, carry):
            boff = base + bi * _B
            pltpu.sync_copy(src_h.at[pl.ds(boff, _B)], si)
            pltpu.sync_copy(dst_h.at[pl.ds(boff, _B)], di)

            stage(0, bufs[0])

            def ring(i, c2):
                ci0 = i * 2
                stage(ci0 + 1, bufs[1])
                finish(ci0, bufs[0])
                stage(ci0 + 2, bufs[0])
                finish(ci0 + 1, bufs[1])
                return c2

            lax.fori_loop(0, (_CPB - 1) // 2, ring, 0)
            finish(_CPB - 1, bufs[0])

            pltpu.sync_copy(ob, out_h.at[pl.ds(boff, _B)])
            return carry

        lax.fori_loop(0, n_blocks, block_body, 0)

    return angle_kernel(distances, vec.reshape(-1), angle_src, angle_dst)
